# baseline (speedup 1.0000x reference)
import functools

import jax
import jax.numpy as jnp
from jax import lax
from jax.experimental import pallas as pl
from jax.experimental.pallas import tpu as pltpu
from jax.experimental.pallas import tpu_sc as plsc

_J, _F, _C, _B = 25, 32768, 64, 16
_NC, _NS = 2, 16
_NW = _NC * _NS
_FPW = _F // _NW
_ROWS = _J * _B
_PR = 200
_CROW = _PR
_PRPAD = 208
_PRPS = 16
_NRT = _PRPAD // _PRPS
_NG = _C // 16


def _mesh():
    return plsc.VectorSubcoreMesh(
        core_axis_name="c", subcore_axis_name="s",
        num_cores=_NC, num_subcores=_NS)


def _zero_rows(ref, nrows):
    z = jnp.zeros((16,), jnp.float32)

    def body(r, _):
        for g in range(8):
            ref[r, pl.ds(g * 16, 16)] = z
        return 0

    lax.fori_loop(0, nrows, body, 0)


def _stage_reduce_store(acc, shared, tmp, racc, out, cid, sid):
    pltpu.sync_copy(acc, shared.at[sid])
    plsc.subcore_barrier()

    @pl.when(sid < _NRT)
    def _():
        rlo = sid * _PRPS
        _zero_rows(racc, _PRPS)
        for t in range(_NS):
            pltpu.sync_copy(shared.at[t, pl.ds(rlo, _PRPS)], tmp)

            def rbody(r, _):
                for g in range(8):
                    racc[r, pl.ds(g * 16, 16)] = (
                        racc[r, pl.ds(g * 16, 16)] + tmp[r, pl.ds(g * 16, 16)])
                return 0

            lax.fori_loop(0, _PRPS, rbody, 0)
        pltpu.sync_copy(racc, out.at[cid, pl.ds(rlo, _PRPS)])


def _count_and_bounds(idxv, cnt, bound):
    for b in range(_B):
        cnt[b] = 0

    def cbody(gi, _):
        segv = idxv[pl.ds(gi * 16, 16)]
        for l in range(16):
            seg = segv[l]
            cnt[seg] = cnt[seg] + 1
        return 0

    lax.fori_loop(0, _FPW // 16, cbody, 0)
    bound[0] = 0
    for b in range(_B):
        bound[b + 1] = bound[b] + cnt[b]


def _pipelined_task_loop(x_hbm, f0, bf, buf0, buf1, sem0, sem1, process):
    nblk = _FPW // bf
    ntask = _J * nblk
    f0r, bfr = f0 >> 1, bf >> 1

    def copy(task, buf, sem):
        j = task // nblk
        blk = lax.rem(task, nblk)
        rb = pl.multiple_of(f0r + blk * bfr, 8)
        return pltpu.async_copy(x_hbm.at[j, pl.ds(rb, bfr)], buf, sem)

    copy(jnp.int32(0), buf0, sem0)

    def body(it, _):
        a = it * 2
        copy(a + 1, buf1, sem1)
        pltpu.make_async_copy(
            x_hbm.at[0, pl.ds(pl.multiple_of(f0r, 8), bfr)],
            buf0, sem0).wait()
        process(a, buf0)

        @pl.when(a + 2 < ntask)
        def _():
            copy(a + 2, buf0, sem0)

        pltpu.make_async_copy(
            x_hbm.at[0, pl.ds(pl.multiple_of(f0r, 8), bfr)],
            buf1, sem1).wait()
        process(a + 1, buf1)
        return 0

    lax.fori_loop(0, ntask // 2, body, 0)


def _block_seg_range(idxv, ib, bf):
    seg_first = idxv[pl.ds(ib, 16)][0]
    seg_last = idxv[pl.ds(ib + (bf - 16), 16)][15]
    return seg_first, seg_last


def _tc_retile(x):
    def ker(a_ref, b_ref, o_ref):
        o_ref[0] = jnp.concatenate([a_ref[0], b_ref[0]], axis=-1)

    return pl.pallas_call(
        ker,
        grid=(_J, _NW),
        in_specs=[
            pl.BlockSpec((1, 512, _C), lambda j, q: (j, 2 * q, 0)),
            pl.BlockSpec((1, 512, _C), lambda j, q: (j, 2 * q + 1, 0)),
        ],
        out_specs=pl.BlockSpec((1, 512, 2 * _C), lambda j, q: (j, q, 0)),
        out_shape=jax.ShapeDtypeStruct((_J, _F // 2, 2 * _C), jnp.float32),
    )(x, x)


def _sc_pass1(x, batch_index):
    bf = 512
    nblk = _FPW // bf

    @functools.partial(
        pl.kernel,
        out_type=jax.ShapeDtypeStruct((_NC, _PRPAD, 2 * _C), jnp.float32),
        mesh=_mesh(),
        scratch_types=[
            pltpu.VMEM((bf // 2, 2 * _C), jnp.float32),
            pltpu.VMEM((bf // 2, 2 * _C), jnp.float32),
            pltpu.VMEM((_PRPAD, 2 * _C), jnp.float32),
            pltpu.VMEM((_FPW,), jnp.int32),
            pltpu.VMEM((_PRPS, 2 * _C), jnp.float32),
            pltpu.VMEM((_PRPS, 2 * _C), jnp.float32),
            pltpu.VMEM_SHARED((_NS, _PRPAD, 2 * _C), jnp.float32),
            pltpu.SMEM((_B,), jnp.int32),
            pltpu.SMEM((_B + 1,), jnp.int32),
            pltpu.SemaphoreType.DMA,
            pltpu.SemaphoreType.DMA,
        ],
    )
    def ker(x_hbm, idx_hbm, out, buf0, buf1, acc, idxv, tmp, racc,
            shared, cnt, bound, sem0, sem1):
        cid = lax.axis_index("c")
        sid = lax.axis_index("s")
        wid = sid * _NC + cid
        f0 = wid * _FPW

        _zero_rows(acc, _PRPAD)
        pltpu.sync_copy(idx_hbm.at[pl.ds(f0, _FPW)], idxv)
        _count_and_bounds(idxv, cnt, bound)

        for b in range(_B):
            v = jnp.full((16,), cnt[b].astype(jnp.float32))
            for g in range(_NG):
                acc[_CROW + b // 2, pl.ds((b % 2) * _C + g * 16, 16)] = v

        zv = jnp.zeros((16,), jnp.float32)

        def process(task, buf):
            j = task // nblk
            blk = lax.rem(task, nblk)
            jpr = j * (_B // 2)
            for half in range(2):
                fb = half * 512 + blk * (bf // 2)
                lhb = half * _C
                seg_first, seg_last = _block_seg_range(idxv, fb, bf // 2)

                def segbody(seg, _):
                    lo = jnp.maximum(bound[seg], fb)
                    hi = jnp.minimum(bound[seg + 1], fb + bf // 2)
                    pr = jpr + (seg >> 1)
                    lb = (seg & 1) * _C

                    @plsc.parallel_loop(lo, hi, unroll=4, carry=(zv,) * _NG)
                    def c(f, c):
                        return tuple(
                            c[g] + buf[f - fb, pl.ds(lhb + g * 16, 16)]
                            for g in range(_NG))

                    for g in range(_NG):
                        acc[pr, pl.ds(lb + g * 16, 16)] = (
                            acc[pr, pl.ds(lb + g * 16, 16)] + c[g])
                    return 0

                lax.fori_loop(seg_first, seg_last + 1, segbody, 0)

        _pipelined_task_loop(x_hbm, f0, bf, buf0, buf1, sem0, sem1, process)
        _stage_reduce_store(acc, shared, tmp, racc, out, cid, sid)

    return ker(x, batch_index)


def _sc_pass2(x, batch_index, gc_packed):
    bf = 512
    nblk = _FPW // bf

    @functools.partial(
        pl.kernel,
        out_type=jax.ShapeDtypeStruct((_NC, _PRPAD, 2 * _C), jnp.float32),
        mesh=_mesh(),
        scratch_types=[
            pltpu.VMEM((bf // 2, 2 * _C), jnp.float32),
            pltpu.VMEM((bf // 2, 2 * _C), jnp.float32),
            pltpu.VMEM((_PRPAD, 2 * _C), jnp.float32),
            pltpu.VMEM((16, 2 * _C), jnp.float32),
            pltpu.VMEM((_FPW,), jnp.int32),
            pltpu.VMEM((_PRPS, 2 * _C), jnp.float32),
            pltpu.VMEM((_PRPS, 2 * _C), jnp.float32),
            pltpu.VMEM_SHARED((_NS, _PRPAD, 2 * _C), jnp.float32),
            pltpu.SMEM((_B,), jnp.int32),
            pltpu.SMEM((_B + 1,), jnp.int32),
            pltpu.SemaphoreType.DMA,
            pltpu.SemaphoreType.DMA,
        ],
    )
    def ker(x_hbm, idx_hbm, gc_hbm, out, buf0, buf1, acc, aux, idxv,
            tmp, racc, shared, cnt, bound, sem0, sem1):
        cid = lax.axis_index("c")
        sid = lax.axis_index("s")
        wid = sid * _NC + cid
        f0 = wid * _FPW

        _zero_rows(acc, _PRPAD)
        pltpu.sync_copy(idx_hbm.at[pl.ds(f0, _FPW)], idxv)
        _count_and_bounds(idxv, cnt, bound)

        zv = jnp.zeros((16,), jnp.float32)
        perms = [lax.iota(jnp.int32, 16) ^ sh for sh in (8, 4, 2, 1)]
        gdims = lax.GatherDimensionNumbers(
            offset_dims=(), collapsed_slice_dims=(0,), start_index_map=(0,))

        def process(task, buf):
            j = task // nblk
            blk = lax.rem(task, nblk)
            jpr = j * (_B // 2)

            @pl.when(blk == 0)
            def _():
                pltpu.sync_copy(gc_hbm.at[pl.ds(jpr, 16)], aux)

            for half in range(2):
                fb = half * 512 + blk * (bf // 2)
                lhb = half * _C
                seg_first, seg_last = _block_seg_range(idxv, fb, bf // 2)

                def segbody(seg, _):
                    lo = jnp.maximum(bound[seg], fb)
                    hi = jnp.minimum(bound[seg + 1], fb + bf // 2)
                    pr = jpr + (seg >> 1)
                    lb = (seg & 1) * _C
                    gg = [aux[seg >> 1, pl.ds(lb + g * 16, 16)]
                          for g in range(_NG)]

                    @plsc.parallel_loop(lo, hi, unroll=8, carry=(zv,) * _NG)
                    def c(f, c):
                        xg = [buf[f - fb, pl.ds(lhb + g * 16, 16)]
                              for g in range(_NG)]
                        prod = xg[0] * gg[0]
                        for g in range(1, _NG):
                            prod = prod + xg[g] * gg[g]
                        for perm in perms:
                            prod = prod + lax.gather(
                                prod, perm[:, None], gdims, slice_sizes=(1,),
                                mode=lax.GatherScatterMode.PROMISE_IN_BOUNDS)
                        gate = 1.0 / (1.0 + jnp.exp(-prod))
                        return tuple(c[g] + gate * xg[g] for g in range(_NG))

                    for g in range(_NG):
                        acc[pr, pl.ds(lb + g * 16, 16)] = (
                            acc[pr, pl.ds(lb + g * 16, 16)] + c[g])
                    return 0

                lax.fori_loop(seg_first, seg_last + 1, segbody, 0)

        _pipelined_task_loop(x_hbm, f0, bf, buf0, buf1, sem0, sem1, process)
        _stage_reduce_store(acc, shared, tmp, racc, out, cid, sid)

    return ker(x, batch_index, gc_packed)


def _divisors(pa):
    cntm = jnp.maximum(pa[_CROW:_CROW + _B // 2, :], 1.0)
    return jnp.concatenate([cntm] * _J, axis=0)


def _tc_mid(part_a, w):
    def ker(pa_ref, w_ref, gc_ref):
        p = pa_ref[0] + pa_ref[1]
        cntm = jnp.maximum(p[_CROW:_CROW + _B // 2, :], 1.0)
        dvs = jnp.concatenate([cntm] * (_J + 1), axis=0)
        means = p / dvs
        wv = w_ref[...]
        z = jnp.zeros((_C, _C), jnp.float32)
        w2 = jnp.concatenate(
            [jnp.concatenate([wv, z], axis=1),
             jnp.concatenate([z, wv], axis=1)], axis=0)
        gc_ref[...] = jnp.tanh(
            jnp.dot(means, w2, preferred_element_type=jnp.float32))

    return pl.pallas_call(
        ker,
        out_shape=jax.ShapeDtypeStruct((_PRPAD, 2 * _C), jnp.float32),
    )(part_a, w)


def _tc_fin(part_b, part_a):
    def ker(pb_ref, pa_ref, out_ref):
        p = pb_ref[0] + pb_ref[1]
        pa = pa_ref[0] + pa_ref[1]
        out_ref[...] = p[:_PR, :] / _divisors(pa)

    return pl.pallas_call(
        ker,
        out_shape=jax.ShapeDtypeStruct((_PR, 2 * _C), jnp.float32),
    )(part_b, part_a)


def kernel(x, batch_index, W):
    idx = batch_index.astype(jnp.int32)
    xt = _tc_retile(x)
    part_a = _sc_pass1(xt, idx)
    gc = _tc_mid(part_a, W)
    part_b = _sc_pass2(xt, idx, gc)
    out = _tc_fin(part_b, part_a)
    return out.reshape(_J, _B, _C)


# speedup vs baseline: 1.6988x; 1.6988x over previous
import functools

import jax
import jax.numpy as jnp
from jax import lax
from jax.experimental import pallas as pl
from jax.experimental.pallas import tpu as pltpu
from jax.experimental.pallas import tpu_sc as plsc

_J, _F, _C, _B = 25, 32768, 64, 16
_NC, _NS = 2, 16
_NW = _NC * _NS
_FPW = _F // _NW
_ROWS = _J * _B
_PR = 200
_CROW = _PR
_PRPAD = 208
_PRPS = 16
_NRT = _PRPAD // _PRPS
_NG = _C // 16


def _mesh():
    return plsc.VectorSubcoreMesh(
        core_axis_name="c", subcore_axis_name="s",
        num_cores=_NC, num_subcores=_NS)


def _zero_rows(ref, nrows):
    z = jnp.zeros((16,), jnp.float32)

    def body(r, _):
        for g in range(8):
            ref[r, pl.ds(g * 16, 16)] = z
        return 0

    lax.fori_loop(0, nrows, body, 0)


def _stage_reduce_store(acc, shared, tmp, racc, out, cid, sid):
    pltpu.sync_copy(acc, shared.at[sid])
    plsc.subcore_barrier()

    @pl.when(sid < _NRT)
    def _():
        rlo = sid * _PRPS
        _zero_rows(racc, _PRPS)
        for t in range(_NS):
            pltpu.sync_copy(shared.at[t, pl.ds(rlo, _PRPS)], tmp)

            def rbody(r, _):
                for g in range(8):
                    racc[r, pl.ds(g * 16, 16)] = (
                        racc[r, pl.ds(g * 16, 16)] + tmp[r, pl.ds(g * 16, 16)])
                return 0

            lax.fori_loop(0, _PRPS, rbody, 0)
        pltpu.sync_copy(racc, out.at[cid, pl.ds(rlo, _PRPS)])


def _count_and_bounds(idxv, cnt, bound):
    for b in range(_B):
        cnt[b] = 0

    def cbody(gi, _):
        segv = idxv[pl.ds(gi * 16, 16)]
        for l in range(16):
            seg = segv[l]
            cnt[seg] = cnt[seg] + 1
        return 0

    lax.fori_loop(0, _FPW // 16, cbody, 0)
    bound[0] = 0
    for b in range(_B):
        bound[b + 1] = bound[b] + cnt[b]


def _pipelined_task_loop(x_hbm, f0, bf, buf0, buf1, sem0, sem1, process):
    nblk = _FPW // bf
    ntask = _J * nblk

    def copy(task, buf, sem):
        j = task // nblk
        blk = lax.rem(task, nblk)
        return pltpu.async_copy(
            x_hbm.at[j, pl.ds(f0 + blk * bf, bf)], buf, sem)

    copy(jnp.int32(0), buf0, sem0)

    def body(it, _):
        a = it * 2
        copy(a + 1, buf1, sem1)
        pltpu.make_async_copy(x_hbm.at[0, pl.ds(f0, bf)], buf0, sem0).wait()
        process(a, buf0)

        @pl.when(a + 2 < ntask)
        def _():
            copy(a + 2, buf0, sem0)

        pltpu.make_async_copy(x_hbm.at[0, pl.ds(f0, bf)], buf1, sem1).wait()
        process(a + 1, buf1)
        return 0

    lax.fori_loop(0, ntask // 2, body, 0)


def _block_seg_range(idxv, ib, bf):
    seg_first = idxv[pl.ds(ib, 16)][0]
    seg_last = idxv[pl.ds(ib + (bf - 16), 16)][15]
    return seg_first, seg_last


def _sc_pass1(x, batch_index):
    bf = 256
    nblk = _FPW // bf

    @functools.partial(
        pl.kernel,
        out_type=jax.ShapeDtypeStruct((_NC, _PRPAD, 2 * _C), jnp.float32),
        mesh=_mesh(),
        scratch_types=[
            pltpu.VMEM((bf, _C), jnp.float32),
            pltpu.VMEM((bf, _C), jnp.float32),
            pltpu.VMEM((_PRPAD, 2 * _C), jnp.float32),
            pltpu.VMEM((_FPW,), jnp.int32),
            pltpu.VMEM((_PRPS, 2 * _C), jnp.float32),
            pltpu.VMEM((_PRPS, 2 * _C), jnp.float32),
            pltpu.VMEM_SHARED((_NS, _PRPAD, 2 * _C), jnp.float32),
            pltpu.SMEM((_B,), jnp.int32),
            pltpu.SMEM((_B + 1,), jnp.int32),
            pltpu.SemaphoreType.DMA,
            pltpu.SemaphoreType.DMA,
        ],
    )
    def ker(x_hbm, idx_hbm, out, buf0, buf1, acc, idxv, tmp, racc,
            shared, cnt, bound, sem0, sem1):
        cid = lax.axis_index("c")
        sid = lax.axis_index("s")
        wid = sid * _NC + cid
        f0 = wid * _FPW

        _zero_rows(acc, _PRPAD)
        pltpu.sync_copy(idx_hbm.at[pl.ds(f0, _FPW)], idxv)
        _count_and_bounds(idxv, cnt, bound)

        for b in range(_B):
            v = jnp.full((16,), cnt[b].astype(jnp.float32))
            for g in range(_NG):
                acc[_CROW + b // 2, pl.ds((b % 2) * _C + g * 16, 16)] = v

        zv = jnp.zeros((16,), jnp.float32)

        def process(task, buf):
            j = task // nblk
            blk = lax.rem(task, nblk)
            jpr = j * (_B // 2)
            ib = blk * bf
            seg_first, seg_last = _block_seg_range(idxv, ib, bf)

            def segbody(seg, _):
                lo = jnp.maximum(bound[seg], ib) - ib
                hi = jnp.minimum(bound[seg + 1], ib + bf) - ib
                pr = jpr + (seg >> 1)
                lb = (seg & 1) * _C

                @plsc.parallel_loop(lo, hi, unroll=4, carry=(zv,) * _NG)
                def c(f, c):
                    return tuple(
                        c[g] + buf[f, pl.ds(g * 16, 16)] for g in range(_NG))
                for g in range(_NG):
                    acc[pr, pl.ds(lb + g * 16, 16)] = (
                        acc[pr, pl.ds(lb + g * 16, 16)] + c[g])
                return 0

            lax.fori_loop(seg_first, seg_last + 1, segbody, 0)

        _pipelined_task_loop(x_hbm, f0, bf, buf0, buf1, sem0, sem1, process)
        _stage_reduce_store(acc, shared, tmp, racc, out, cid, sid)

    return ker(x, batch_index)


def _sc_pass2(x, batch_index, gc_packed):
    bf = 256
    nblk = _FPW // bf

    @functools.partial(
        pl.kernel,
        out_type=jax.ShapeDtypeStruct((_NC, _PRPAD, 2 * _C), jnp.float32),
        mesh=_mesh(),
        scratch_types=[
            pltpu.VMEM((bf, _C), jnp.float32),
            pltpu.VMEM((bf, _C), jnp.float32),
            pltpu.VMEM((_PRPAD, 2 * _C), jnp.float32),
            pltpu.VMEM((16, 2 * _C), jnp.float32),
            pltpu.VMEM((_FPW,), jnp.int32),
            pltpu.VMEM((_PRPS, 2 * _C), jnp.float32),
            pltpu.VMEM((_PRPS, 2 * _C), jnp.float32),
            pltpu.VMEM_SHARED((_NS, _PRPAD, 2 * _C), jnp.float32),
            pltpu.SMEM((_B,), jnp.int32),
            pltpu.SMEM((_B + 1,), jnp.int32),
            pltpu.SemaphoreType.DMA,
            pltpu.SemaphoreType.DMA,
        ],
    )
    def ker(x_hbm, idx_hbm, gc_hbm, out, buf0, buf1, acc, aux, idxv,
            tmp, racc, shared, cnt, bound, sem0, sem1):
        cid = lax.axis_index("c")
        sid = lax.axis_index("s")
        wid = sid * _NC + cid
        f0 = wid * _FPW

        _zero_rows(acc, _PRPAD)
        pltpu.sync_copy(idx_hbm.at[pl.ds(f0, _FPW)], idxv)
        _count_and_bounds(idxv, cnt, bound)

        zv = jnp.zeros((16,), jnp.float32)
        perms = [lax.iota(jnp.int32, 16) ^ sh for sh in (8, 4, 2, 1)]
        gdims = lax.GatherDimensionNumbers(
            offset_dims=(), collapsed_slice_dims=(0,), start_index_map=(0,))

        def process(task, buf):
            j = task // nblk
            blk = lax.rem(task, nblk)
            jpr = j * (_B // 2)
            ib = blk * bf

            @pl.when(blk == 0)
            def _():
                pltpu.sync_copy(gc_hbm.at[pl.ds(jpr, 16)], aux)

            seg_first, seg_last = _block_seg_range(idxv, ib, bf)

            def segbody(seg, _):
                lo = jnp.maximum(bound[seg], ib) - ib
                hi = jnp.minimum(bound[seg + 1], ib + bf) - ib
                pr = jpr + (seg >> 1)
                lb = (seg & 1) * _C
                gg = [aux[seg >> 1, pl.ds(lb + g * 16, 16)]
                      for g in range(_NG)]

                @plsc.parallel_loop(lo, hi, unroll=8, carry=(zv,) * _NG)
                def c(f, c):
                    xg = [buf[f, pl.ds(g * 16, 16)] for g in range(_NG)]
                    prod = xg[0] * gg[0]
                    for g in range(1, _NG):
                        prod = prod + xg[g] * gg[g]
                    for perm in perms:
                        prod = prod + lax.gather(
                            prod, perm[:, None], gdims, slice_sizes=(1,),
                            mode=lax.GatherScatterMode.PROMISE_IN_BOUNDS)
                    gate = 1.0 / (1.0 + jnp.exp(-prod))
                    return tuple(c[g] + gate * xg[g] for g in range(_NG))
                for g in range(_NG):
                    acc[pr, pl.ds(lb + g * 16, 16)] = (
                        acc[pr, pl.ds(lb + g * 16, 16)] + c[g])
                return 0

            lax.fori_loop(seg_first, seg_last + 1, segbody, 0)

        _pipelined_task_loop(x_hbm, f0, bf, buf0, buf1, sem0, sem1, process)
        _stage_reduce_store(acc, shared, tmp, racc, out, cid, sid)

    return ker(x, batch_index, gc_packed)


def _divisors(pa):
    cntm = jnp.maximum(pa[_CROW:_CROW + _B // 2, :], 1.0)
    return jnp.concatenate([cntm] * _J, axis=0)


def _tc_mid(part_a, w):
    def ker(pa_ref, w_ref, gc_ref):
        p = pa_ref[0] + pa_ref[1]
        cntm = jnp.maximum(p[_CROW:_CROW + _B // 2, :], 1.0)
        dvs = jnp.concatenate([cntm] * (_J + 1), axis=0)
        means = p / dvs
        wv = w_ref[...]
        z = jnp.zeros((_C, _C), jnp.float32)
        w2 = jnp.concatenate(
            [jnp.concatenate([wv, z], axis=1),
             jnp.concatenate([z, wv], axis=1)], axis=0)
        gc_ref[...] = jnp.tanh(
            jnp.dot(means, w2, preferred_element_type=jnp.float32))

    return pl.pallas_call(
        ker,
        out_shape=jax.ShapeDtypeStruct((_PRPAD, 2 * _C), jnp.float32),
    )(part_a, w)


def _tc_fin(part_b, part_a):
    def ker(pb_ref, pa_ref, out_ref):
        p = pb_ref[0] + pb_ref[1]
        pa = pa_ref[0] + pa_ref[1]
        out_ref[...] = p[:_PR, :] / _divisors(pa)

    return pl.pallas_call(
        ker,
        out_shape=jax.ShapeDtypeStruct((_PR, 2 * _C), jnp.float32),
    )(part_b, part_a)


def kernel(x, batch_index, W):
    idx = batch_index.astype(jnp.int32)
    part_a = _sc_pass1(x, idx)
    gc = _tc_mid(part_a, W)
    part_b = _sc_pass2(x, idx, gc)
    out = _tc_fin(part_b, part_a)
    return out.reshape(_J, _B, _C)
